# Initial kernel scaffold; baseline (speedup 1.0000x reference)
import jax
import jax.numpy as jnp
from jax import lax
from jax.experimental import pallas as pl
from jax.experimental.pallas import tpu as pltpu
from jax.experimental.pallas import tpu_sc as plsc

N_NODES = 10000
NP = 10016
E2 = 640000
EH = 320000
Q = 8
NC, NS, NW = 2, 16, 32
EPT = E2 // NW
W = 1000
NWIN = EPT // W
CH = 125
NCH = W // CH
NPT = NP // NS
NPO = NP // NW

F32 = jnp.float32
I32 = jnp.int32

_LOG1P_COEF = (0.9999947246324665, -0.49984485381321225, 0.3317124871811307,
               -0.2413461885986923, 0.17221571318743972, -0.10745355469223546,
               0.052375598963768255, -0.01786311035930909, 0.0037034338062475664,
               -0.0003470312864672028)


def _it():
    return lax.iota(I32, 16)


def _perm(v, p):
    return jnp.take(v, p, mode="promise_in_bounds")


def _hmax(v):
    it = _it()
    for k in (1, 2, 4):
        v = jnp.maximum(v, _perm(v, it ^ k))
    return v


def _hsum(v):
    it = _it()
    for k in (1, 2, 4):
        v = v + _perm(v, it ^ k)
    return v


def _log1p(u):
    q = u * _LOG1P_COEF[9]
    for k in range(8, 0, -1):
        q = (q + _LOG1P_COEF[k]) * u
    return (q + _LOG1P_COEF[0]) * u


def _pair(i):
    it = _it()
    return 2 * i + (it >> 3), it & 7


def _k0_body(msg, dst3, cvec, ns_out, deg_out,
             didx, mbuf, lbuf, obuf, zbuf, cbuf, ns_sh, deg_sh):
    cid = lax.axis_index("c")
    sid = lax.axis_index("s")
    wid = cid * NS + sid
    pltpu.sync_copy(cvec, cbuf)
    cc = cbuf[...]
    colp = _it() & 7

    def zinit(i, a):
        row, col = _pair(i)
        plsc.store_scatter(zbuf, [row, col], jnp.zeros((16,), F32))
        return a
    lax.fori_loop(0, NPT // 2, zinit, 0)

    def oinit(i, a):
        row, col = _pair(i)
        plsc.store_scatter(obuf, [row, col], jnp.ones((16,), F32))
        return a
    lax.fori_loop(0, W // 2, oinit, 0)

    n0 = sid * NPT
    pltpu.sync_copy(zbuf, ns_sh.at[pl.ds(n0, NPT)])
    pltpu.sync_copy(zbuf, deg_sh.at[pl.ds(n0, NPT)])
    plsc.subcore_barrier()

    e0 = wid * EPT

    def win(w, a):
        g = wid * NWIN + w
        pltpu.sync_copy(dst3.at[g], didx)
        pltpu.sync_copy(msg.at[pl.ds(e0 + w * W, W)], mbuf)

        def pair(i, b):
            row, col = _pair(i)
            m2 = plsc.load_gather(mbuf, [row, col])
            plsc.store_scatter(lbuf, [row, col], _log1p(cc * m2))
            return b
        lax.fori_loop(0, W // 2, pair, 0)
        for j in range(NCH):
            sl = pl.ds(j * CH, CH)
            pltpu.sync_copy(lbuf.at[sl], ns_sh.at[didx.at[j]], add=True)
            pltpu.sync_copy(obuf.at[sl], deg_sh.at[didx.at[j]], add=True)
        return a
    lax.fori_loop(0, NWIN, win, 0)

    plsc.subcore_barrier()
    pltpu.sync_copy(ns_sh.at[pl.ds(n0, NPT)], ns_out.at[cid].at[pl.ds(n0, NPT)])
    pltpu.sync_copy(deg_sh.at[pl.ds(n0, NPT)], deg_out.at[cid].at[pl.ds(n0, NPT)])


def _gs_body(msg, nsp, degp, src3, dst3, cvec, bovn, newmsg, nsp_out,
             sidx, didx, ebuf, mbuf, obuf, lbuf,
             nsA, nsB, dgA, dgB, pbuf, htmp, hall, cbuf, bbuf,
             ns_sh, e_sh, h_sh):
    cid = lax.axis_index("c")
    sid = lax.axis_index("s")
    wid = cid * NS + sid
    pltpu.sync_copy(cvec, cbuf)
    pltpu.sync_copy(bovn, bbuf)
    cc = cbuf[...]
    it = _it()

    n0 = sid * NPT
    pltpu.sync_copy(nsp.at[0].at[pl.ds(n0, NPT)], nsA)
    pltpu.sync_copy(nsp.at[1].at[pl.ds(n0, NPT)], nsB)
    pltpu.sync_copy(degp.at[0].at[pl.ds(n0, NPT)], dgA)
    pltpu.sync_copy(degp.at[1].at[pl.ds(n0, NPT)], dgB)

    def hstep(i, hacc):
        row, col = _pair(i)
        a = plsc.load_gather(nsA, [row, col]) + plsc.load_gather(nsB, [row, col])
        dv = plsc.load_gather(dgA, [row, col]) + plsc.load_gather(dgB, [row, col])
        p = jnp.exp(a - _hmax(a))
        psi = p / _hsum(p)
        plsc.store_scatter(pbuf, [row, col], p)
        plsc.store_scatter(nsB, [row, col], jnp.zeros((16,), F32))
        return hacc + dv * psi
    hacc = lax.fori_loop(0, NPT // 2, hstep, jnp.zeros((16,), F32))
    htmp[...] = hacc
    pltpu.sync_copy(htmp, h_sh.at[sid])
    plsc.subcore_barrier()
    pltpu.sync_copy(h_sh, hall)
    hs = jnp.zeros((16,), F32)
    for k in range(NS):
        hs = hs + hall[k, :]
    hs = hs + _perm(hs, it ^ 8)
    ehv = jnp.exp(-bbuf[...] * hs)

    def estep(i, a):
        row, col = _pair(i)
        p = plsc.load_gather(pbuf, [row, col])
        plsc.store_scatter(pbuf, [row, col], p * ehv)
        return a
    lax.fori_loop(0, NPT // 2, estep, 0)
    pltpu.sync_copy(pbuf, e_sh.at[pl.ds(n0, NPT)])
    pltpu.sync_copy(nsB, ns_sh.at[pl.ds(n0, NPT)])
    plsc.subcore_barrier()

    e0 = wid * EPT
    rev0 = jnp.where(wid < NS, e0 + EH, e0 - EH)

    def win(w, a):
        g = wid * NWIN + w
        pltpu.sync_copy(src3.at[g], sidx)
        pltpu.sync_copy(dst3.at[g], didx)
        pltpu.sync_copy(msg.at[pl.ds(rev0 + w * W, W)], mbuf)
        for j in range(NCH):
            pltpu.sync_copy(e_sh.at[sidx.at[j]], ebuf.at[pl.ds(j * CH, CH)])

        def pair(i, b):
            row, col = _pair(i)
            ev = plsc.load_gather(ebuf, [row, col])
            mv = plsc.load_gather(mbuf, [row, col])
            wv = ev / (cc * mv + 1.0)
            nm = wv / _hsum(wv)
            plsc.store_scatter(obuf, [row, col], nm)
            plsc.store_scatter(lbuf, [row, col], _log1p(cc * nm))
            return b
        lax.fori_loop(0, W // 2, pair, 0)
        pltpu.sync_copy(obuf, newmsg.at[pl.ds(e0 + w * W, W)])
        for j in range(NCH):
            pltpu.sync_copy(lbuf.at[pl.ds(j * CH, CH)],
                            ns_sh.at[didx.at[j]], add=True)
        return a
    lax.fori_loop(0, NWIN, win, 0)

    plsc.subcore_barrier()
    pltpu.sync_copy(ns_sh.at[pl.ds(n0, NPT)], nsp_out.at[cid].at[pl.ds(n0, NPT)])


def _marg_body(nsp, degp, bovn, psi_out,
               nsA, nsB, dgA, dgB, obuf, htmp, hall, bbuf, h_sh):
    cid = lax.axis_index("c")
    sid = lax.axis_index("s")
    wid = cid * NS + sid
    pltpu.sync_copy(bovn, bbuf)
    it = _it()

    n0 = sid * NPT
    pltpu.sync_copy(nsp.at[0].at[pl.ds(n0, NPT)], nsA)
    pltpu.sync_copy(nsp.at[1].at[pl.ds(n0, NPT)], nsB)
    pltpu.sync_copy(degp.at[0].at[pl.ds(n0, NPT)], dgA)
    pltpu.sync_copy(degp.at[1].at[pl.ds(n0, NPT)], dgB)

    def hstep(i, hacc):
        row, col = _pair(i)
        a = plsc.load_gather(nsA, [row, col]) + plsc.load_gather(nsB, [row, col])
        dv = plsc.load_gather(dgA, [row, col]) + plsc.load_gather(dgB, [row, col])
        p = jnp.exp(a - _hmax(a))
        psi = p / _hsum(p)
        return hacc + dv * psi
    hacc = lax.fori_loop(0, NPT // 2, hstep, jnp.zeros((16,), F32))
    htmp[...] = hacc
    pltpu.sync_copy(htmp, h_sh.at[sid])
    plsc.subcore_barrier()
    pltpu.sync_copy(h_sh, hall)
    hs = jnp.zeros((16,), F32)
    for k in range(NS):
        hs = hs + hall[k, :]
    hs = hs + _perm(hs, it ^ 8)
    hv = bbuf[...] * hs

    g0 = wid * NPO
    pltpu.sync_copy(nsp.at[0].at[pl.ds(g0, NPO)], nsA.at[pl.ds(0, NPO)])
    pltpu.sync_copy(nsp.at[1].at[pl.ds(g0, NPO)], nsB.at[pl.ds(0, NPO)])

    def ostep(i, a):
        row, col = _pair(i)
        v = (plsc.load_gather(nsA, [row, col])
             + plsc.load_gather(nsB, [row, col]) - hv)
        p = jnp.exp(v - _hmax(v))
        psi = p / _hsum(p)
        plsc.store_scatter(obuf, [row, col], psi)
        return a
    lax.fori_loop(0, (NPO + 1) // 2, ostep, 0)
    pltpu.sync_copy(obuf.at[pl.ds(0, NPO)], psi_out.at[pl.ds(g0, NPO)])


def _mesh():
    return plsc.VectorSubcoreMesh(core_axis_name="c", subcore_axis_name="s")


@jax.jit
def _run(msg0, src3, dst3, cvec, bovn):
    k0 = pl.kernel(
        _k0_body,
        out_type=(jax.ShapeDtypeStruct((NC, NP, Q), F32),
                  jax.ShapeDtypeStruct((NC, NP, Q), F32)),
        mesh=_mesh(),
        scratch_types=[
            pltpu.VMEM((NCH, CH), I32),
            pltpu.VMEM((W, Q), F32),
            pltpu.VMEM((W, Q), F32),
            pltpu.VMEM((W, Q), F32),
            pltpu.VMEM((NPT, Q), F32),
            pltpu.VMEM((16,), F32),
            pltpu.VMEM_SHARED((NP, Q), F32),
            pltpu.VMEM_SHARED((NP, Q), F32),
        ],
    )
    gs = pl.kernel(
        _gs_body,
        out_type=(jax.ShapeDtypeStruct((E2, Q), F32),
                  jax.ShapeDtypeStruct((NC, NP, Q), F32)),
        mesh=_mesh(),
        scratch_types=[
            pltpu.VMEM((NCH, CH), I32),
            pltpu.VMEM((NCH, CH), I32),
            pltpu.VMEM((W, Q), F32),
            pltpu.VMEM((W, Q), F32),
            pltpu.VMEM((W, Q), F32),
            pltpu.VMEM((W, Q), F32),
            pltpu.VMEM((NPT, Q), F32),
            pltpu.VMEM((NPT, Q), F32),
            pltpu.VMEM((NPT, Q), F32),
            pltpu.VMEM((NPT, Q), F32),
            pltpu.VMEM((NPT, Q), F32),
            pltpu.VMEM((16,), F32),
            pltpu.VMEM((NS, 16), F32),
            pltpu.VMEM((16,), F32),
            pltpu.VMEM((16,), F32),
            pltpu.VMEM_SHARED((NP, Q), F32),
            pltpu.VMEM_SHARED((NP, Q), F32),
            pltpu.VMEM_SHARED((NS, 16), F32),
        ],
    )
    marg = pl.kernel(
        _marg_body,
        out_type=jax.ShapeDtypeStruct((NP, Q), F32),
        mesh=_mesh(),
        scratch_types=[
            pltpu.VMEM((NPT, Q), F32),
            pltpu.VMEM((NPT, Q), F32),
            pltpu.VMEM((NPT, Q), F32),
            pltpu.VMEM((NPT, Q), F32),
            pltpu.VMEM((NPO + 1, Q), F32),
            pltpu.VMEM((16,), F32),
            pltpu.VMEM((NS, 16), F32),
            pltpu.VMEM((16,), F32),
            pltpu.VMEM_SHARED((NS, 16), F32),
        ],
    )
    nsp, degp = k0(msg0, dst3, cvec)
    msg = msg0
    for _ in range(5):
        msg, nsp = gs(msg, nsp, degp, src3, dst3, cvec, bovn)
    psi_pad = marg(nsp, degp, bovn)
    return msg, psi_pad[:N_NODES]


def kernel(edge_index, num_nodes, beta, message_map_init):
    src = jnp.concatenate([edge_index[0], edge_index[1]]).astype(I32)
    dst = jnp.concatenate([edge_index[1], edge_index[0]]).astype(I32)
    src3 = src.reshape(E2 // W, NCH, CH)
    dst3 = dst.reshape(E2 // W, NCH, CH)
    beta = jnp.asarray(beta, F32)
    cvec = jnp.full((16,), jnp.exp(beta) - 1.0, F32)
    bovn = jnp.full((16,), beta / jnp.asarray(num_nodes, F32), F32)
    return _run(message_map_init, src3, dst3, cvec, bovn)


# speedup vs baseline: 7.3021x; 7.3021x over previous
import jax
import jax.numpy as jnp
from jax import lax
from jax.experimental import pallas as pl
from jax.experimental.pallas import tpu as pltpu
from jax.experimental.pallas import tpu_sc as plsc

N_NODES = 10000
NP = 10240
E2 = 640000
EH = 320000
Q = 8
NC, NS, NW = 2, 16, 32
EPT = E2 // NW
W = 1000
NWIN = EPT // W
CH = 125
NCH = W // CH
NPT = NP // NS
NPO = NP // NW

F32 = jnp.float32
I32 = jnp.int32

_LOG1P_COEF = (0.9999947246324665, -0.49984485381321225, 0.3317124871811307,
               -0.2413461885986923, 0.17221571318743972, -0.10745355469223546,
               0.052375598963768255, -0.01786311035930909, 0.0037034338062475664,
               -0.0003470312864672028)


def _it():
    return lax.iota(I32, 16)


def _perm(v, p):
    dnums = lax.GatherDimensionNumbers(
        offset_dims=(), collapsed_slice_dims=(0,), start_index_map=(0,))
    return lax.gather(v, p[:, None], dnums, (1,),
                      mode=lax.GatherScatterMode.PROMISE_IN_BOUNDS)


def _hmax(v):
    it = _it()
    for k in (1, 2, 4):
        v = jnp.maximum(v, _perm(v, it ^ k))
    return v


def _hsum(v):
    it = _it()
    for k in (1, 2, 4):
        v = v + _perm(v, it ^ k)
    return v


def _log1p(u):
    q = u * _LOG1P_COEF[9]
    for k in range(8, 0, -1):
        q = (q + _LOG1P_COEF[k]) * u
    return (q + _LOG1P_COEF[0]) * u


def _pair(i):
    it = _it()
    return 2 * i + (it >> 3), it & 7


def _k0_body(msg, dst3, cvec, ns_out, deg_out,
             didx, mbuf, lbuf, obuf, zbuf, cbuf, ns_sh, deg_sh):
    cid = lax.axis_index("c")
    sid = lax.axis_index("s")
    wid = cid * NS + sid
    pltpu.sync_copy(cvec, cbuf)
    cc = cbuf[...]
    colp = _it() & 7

    def zinit(i, a):
        row, col = _pair(i)
        plsc.store_scatter(zbuf, [row, col], jnp.zeros((16,), F32))
        return a
    lax.fori_loop(0, NPT // 2, zinit, 0)

    def oinit(i, a):
        row, col = _pair(i)
        plsc.store_scatter(obuf, [row, col], jnp.ones((16,), F32))
        return a
    lax.fori_loop(0, W // 2, oinit, 0)

    n0 = sid * NPT
    pltpu.sync_copy(zbuf, ns_sh.at[pl.ds(n0, NPT)])
    pltpu.sync_copy(zbuf, deg_sh.at[pl.ds(n0, NPT)])
    plsc.subcore_barrier()

    e0 = wid * EPT

    def win(w, a):
        g = wid * NWIN + w
        pltpu.sync_copy(dst3.at[g], didx)
        pltpu.sync_copy(msg.at[pl.ds(e0 + w * W, W)], mbuf)

        def pair(i, b):
            row, col = _pair(i)
            m2 = plsc.load_gather(mbuf, [row, col])
            plsc.store_scatter(lbuf, [row, col], _log1p(cc * m2))
            return b
        lax.fori_loop(0, W // 2, pair, 0)
        for j in range(NCH):
            sl = pl.ds(j * CH, CH)
            pltpu.sync_copy(lbuf.at[sl], ns_sh.at[didx.at[j]], add=True)
            pltpu.sync_copy(obuf.at[sl], deg_sh.at[didx.at[j]], add=True)
        return a
    lax.fori_loop(0, NWIN, win, 0)

    plsc.subcore_barrier()
    pltpu.sync_copy(ns_sh.at[pl.ds(n0, NPT)], ns_out.at[cid].at[pl.ds(n0, NPT)])
    pltpu.sync_copy(deg_sh.at[pl.ds(n0, NPT)], deg_out.at[cid].at[pl.ds(n0, NPT)])


def _gs_body(msg, nsp, degp, src3, dst3, cvec, bovn, newmsg, nsp_out,
             sidx, didx, ebuf, mbuf, obuf, lbuf,
             nsA, nsB, dgA, dgB, pbuf, htmp, hall, cbuf, bbuf,
             ns_sh, e_sh, h_sh):
    cid = lax.axis_index("c")
    sid = lax.axis_index("s")
    wid = cid * NS + sid
    pltpu.sync_copy(cvec, cbuf)
    pltpu.sync_copy(bovn, bbuf)
    cc = cbuf[...]
    it = _it()

    n0 = sid * NPT
    pltpu.sync_copy(nsp.at[0].at[pl.ds(n0, NPT)], nsA)
    pltpu.sync_copy(nsp.at[1].at[pl.ds(n0, NPT)], nsB)
    pltpu.sync_copy(degp.at[0].at[pl.ds(n0, NPT)], dgA)
    pltpu.sync_copy(degp.at[1].at[pl.ds(n0, NPT)], dgB)

    def hstep(i, hacc):
        row, col = _pair(i)
        a = plsc.load_gather(nsA, [row, col]) + plsc.load_gather(nsB, [row, col])
        dv = plsc.load_gather(dgA, [row, col]) + plsc.load_gather(dgB, [row, col])
        p = jnp.exp(a - _hmax(a))
        psi = p / _hsum(p)
        plsc.store_scatter(pbuf, [row, col], p)
        plsc.store_scatter(nsB, [row, col], jnp.zeros((16,), F32))
        return hacc + dv * psi
    hacc = lax.fori_loop(0, NPT // 2, hstep, jnp.zeros((16,), F32))
    htmp[...] = hacc
    pltpu.sync_copy(htmp, h_sh.at[pl.ds(sid * 16, 16)])
    plsc.subcore_barrier()
    pltpu.sync_copy(h_sh, hall)
    hs = jnp.zeros((16,), F32)
    for k in range(NS):
        hs = hs + hall[pl.ds(k * 16, 16)]
    hs = hs + _perm(hs, it ^ 8)
    ehv = jnp.exp(-bbuf[...] * hs)

    def estep(i, a):
        row, col = _pair(i)
        p = plsc.load_gather(pbuf, [row, col])
        plsc.store_scatter(pbuf, [row, col], p * ehv)
        return a
    lax.fori_loop(0, NPT // 2, estep, 0)
    pltpu.sync_copy(pbuf, e_sh.at[pl.ds(n0, NPT)])
    pltpu.sync_copy(nsB, ns_sh.at[pl.ds(n0, NPT)])
    plsc.subcore_barrier()

    e0 = wid * EPT
    rev0 = jnp.where(wid < NS, e0 + EH, e0 - EH)

    def win(w, a):
        g = wid * NWIN + w
        pltpu.sync_copy(src3.at[g], sidx)
        pltpu.sync_copy(dst3.at[g], didx)
        pltpu.sync_copy(msg.at[pl.ds(rev0 + w * W, W)], mbuf)
        for j in range(NCH):
            pltpu.sync_copy(e_sh.at[sidx.at[j]], ebuf.at[pl.ds(j * CH, CH)])

        def pair(i, b):
            row, col = _pair(i)
            ev = plsc.load_gather(ebuf, [row, col])
            mv = plsc.load_gather(mbuf, [row, col])
            wv = ev / (cc * mv + 1.0)
            nm = wv / _hsum(wv)
            plsc.store_scatter(obuf, [row, col], nm)
            plsc.store_scatter(lbuf, [row, col], _log1p(cc * nm))
            return b
        lax.fori_loop(0, W // 2, pair, 0)
        pltpu.sync_copy(obuf, newmsg.at[pl.ds(e0 + w * W, W)])
        for j in range(NCH):
            pltpu.sync_copy(lbuf.at[pl.ds(j * CH, CH)],
                            ns_sh.at[didx.at[j]], add=True)
        return a
    lax.fori_loop(0, NWIN, win, 0)

    plsc.subcore_barrier()
    pltpu.sync_copy(ns_sh.at[pl.ds(n0, NPT)], nsp_out.at[cid].at[pl.ds(n0, NPT)])


def _marg_body(nsp, degp, bovn, psi_out,
               nsA, nsB, dgA, dgB, obuf, htmp, hall, bbuf, h_sh):
    cid = lax.axis_index("c")
    sid = lax.axis_index("s")
    wid = cid * NS + sid
    pltpu.sync_copy(bovn, bbuf)
    it = _it()

    n0 = sid * NPT
    pltpu.sync_copy(nsp.at[0].at[pl.ds(n0, NPT)], nsA)
    pltpu.sync_copy(nsp.at[1].at[pl.ds(n0, NPT)], nsB)
    pltpu.sync_copy(degp.at[0].at[pl.ds(n0, NPT)], dgA)
    pltpu.sync_copy(degp.at[1].at[pl.ds(n0, NPT)], dgB)

    def hstep(i, hacc):
        row, col = _pair(i)
        a = plsc.load_gather(nsA, [row, col]) + plsc.load_gather(nsB, [row, col])
        dv = plsc.load_gather(dgA, [row, col]) + plsc.load_gather(dgB, [row, col])
        p = jnp.exp(a - _hmax(a))
        psi = p / _hsum(p)
        return hacc + dv * psi
    hacc = lax.fori_loop(0, NPT // 2, hstep, jnp.zeros((16,), F32))
    htmp[...] = hacc
    pltpu.sync_copy(htmp, h_sh.at[pl.ds(sid * 16, 16)])
    plsc.subcore_barrier()
    pltpu.sync_copy(h_sh, hall)
    hs = jnp.zeros((16,), F32)
    for k in range(NS):
        hs = hs + hall[pl.ds(k * 16, 16)]
    hs = hs + _perm(hs, it ^ 8)
    hv = bbuf[...] * hs

    g0 = wid * NPO
    pltpu.sync_copy(nsp.at[0].at[pl.ds(g0, NPO)], nsA.at[pl.ds(0, NPO)])
    pltpu.sync_copy(nsp.at[1].at[pl.ds(g0, NPO)], nsB.at[pl.ds(0, NPO)])

    def ostep(i, a):
        row, col = _pair(i)
        v = (plsc.load_gather(nsA, [row, col])
             + plsc.load_gather(nsB, [row, col]) - hv)
        p = jnp.exp(v - _hmax(v))
        psi = p / _hsum(p)
        plsc.store_scatter(obuf, [row, col], psi)
        return a
    lax.fori_loop(0, NPO // 2, ostep, 0)
    pltpu.sync_copy(obuf, psi_out.at[pl.ds(g0, NPO)])


def _mesh():
    return plsc.VectorSubcoreMesh(core_axis_name="c", subcore_axis_name="s")


_CPARAMS = pltpu.CompilerParams(needs_layout_passes=False, use_tc_tiling_on_sc=False)


@jax.jit
def _run(msg0, src3, dst3, cvec, bovn):
    k0 = pl.kernel(
        _k0_body,
        out_type=(jax.ShapeDtypeStruct((NC, NP, Q), F32),
                  jax.ShapeDtypeStruct((NC, NP, Q), F32)),
        mesh=_mesh(),
        compiler_params=_CPARAMS,
        scratch_types=[
            pltpu.VMEM((NCH, CH), I32),
            pltpu.VMEM((W, Q), F32),
            pltpu.VMEM((W, Q), F32),
            pltpu.VMEM((W, Q), F32),
            pltpu.VMEM((NPT, Q), F32),
            pltpu.VMEM((16,), F32),
            pltpu.VMEM_SHARED((NP, Q), F32),
            pltpu.VMEM_SHARED((NP, Q), F32),
        ],
    )
    gs = pl.kernel(
        _gs_body,
        out_type=(jax.ShapeDtypeStruct((E2, Q), F32),
                  jax.ShapeDtypeStruct((NC, NP, Q), F32)),
        mesh=_mesh(),
        compiler_params=_CPARAMS,
        scratch_types=[
            pltpu.VMEM((NCH, CH), I32),
            pltpu.VMEM((NCH, CH), I32),
            pltpu.VMEM((W, Q), F32),
            pltpu.VMEM((W, Q), F32),
            pltpu.VMEM((W, Q), F32),
            pltpu.VMEM((W, Q), F32),
            pltpu.VMEM((NPT, Q), F32),
            pltpu.VMEM((NPT, Q), F32),
            pltpu.VMEM((NPT, Q), F32),
            pltpu.VMEM((NPT, Q), F32),
            pltpu.VMEM((NPT, Q), F32),
            pltpu.VMEM((16,), F32),
            pltpu.VMEM((NS * 16,), F32),
            pltpu.VMEM((16,), F32),
            pltpu.VMEM((16,), F32),
            pltpu.VMEM_SHARED((NP, Q), F32),
            pltpu.VMEM_SHARED((NP, Q), F32),
            pltpu.VMEM_SHARED((NS * 16,), F32),
        ],
    )
    marg = pl.kernel(
        _marg_body,
        out_type=jax.ShapeDtypeStruct((NP, Q), F32),
        mesh=_mesh(),
        compiler_params=_CPARAMS,
        scratch_types=[
            pltpu.VMEM((NPT, Q), F32),
            pltpu.VMEM((NPT, Q), F32),
            pltpu.VMEM((NPT, Q), F32),
            pltpu.VMEM((NPT, Q), F32),
            pltpu.VMEM((NPO, Q), F32),
            pltpu.VMEM((16,), F32),
            pltpu.VMEM((NS * 16,), F32),
            pltpu.VMEM((16,), F32),
            pltpu.VMEM_SHARED((NS * 16,), F32),
        ],
    )
    nsp, degp = k0(msg0, dst3, cvec)
    msg = msg0
    for _ in range(5):
        msg, nsp = gs(msg, nsp, degp, src3, dst3, cvec, bovn)
    psi_pad = marg(nsp, degp, bovn)
    return msg, psi_pad[:N_NODES]


def kernel(edge_index, num_nodes, beta, message_map_init):
    src = jnp.concatenate([edge_index[0], edge_index[1]]).astype(I32)
    dst = jnp.concatenate([edge_index[1], edge_index[0]]).astype(I32)
    src3 = src.reshape(E2 // W, NCH, CH)
    dst3 = dst.reshape(E2 // W, NCH, CH)
    beta = jnp.asarray(beta, F32)
    cvec = jnp.full((16,), jnp.exp(beta) - 1.0, F32)
    bovn = jnp.full((16,), beta / jnp.asarray(num_nodes, F32), F32)
    return _run(message_map_init, src3, dst3, cvec, bovn)


# speedup vs baseline: 7.5625x; 1.0357x over previous
import jax
import jax.numpy as jnp
from jax import lax
from jax.experimental import pallas as pl
from jax.experimental.pallas import tpu as pltpu
from jax.experimental.pallas import tpu_sc as plsc

N_NODES = 10000
NP = 10240
E2 = 640000
EH = 320000
Q = 8
NC, NS, NW = 2, 16, 32
EPT = E2 // NW
W = 1000
NWIN = EPT // W
CH = 125
NCH = W // CH
NPT = NP // NS
NPO = NP // NW

F32 = jnp.float32
I32 = jnp.int32

_LOG1P_COEF = (0.9999947246324665, -0.49984485381321225, 0.3317124871811307,
               -0.2413461885986923, 0.17221571318743972, -0.10745355469223546,
               0.052375598963768255, -0.01786311035930909, 0.0037034338062475664,
               -0.0003470312864672028)


def _it():
    return lax.iota(I32, 16)


def _perm(v, p):
    dnums = lax.GatherDimensionNumbers(
        offset_dims=(), collapsed_slice_dims=(0,), start_index_map=(0,))
    return lax.gather(v, p[:, None], dnums, (1,),
                      mode=lax.GatherScatterMode.PROMISE_IN_BOUNDS)


def _hmax(v):
    it = _it()
    for k in (1, 2, 4):
        v = jnp.maximum(v, _perm(v, it ^ k))
    return v


def _hsum(v):
    it = _it()
    for k in (1, 2, 4):
        v = v + _perm(v, it ^ k)
    return v


def _log1p(u):
    q = u * _LOG1P_COEF[9]
    for k in range(8, 0, -1):
        q = (q + _LOG1P_COEF[k]) * u
    return (q + _LOG1P_COEF[0]) * u


def _pair(i):
    it = _it()
    return 2 * i + (it >> 3), it & 7


def _k0_body(msg, dst3, cvec, ns_out, deg_out,
             didx, mbuf, lbuf, obuf, zbuf, cbuf, ns_sh, deg_sh):
    cid = lax.axis_index("c")
    sid = lax.axis_index("s")
    wid = cid * NS + sid
    pltpu.sync_copy(cvec, cbuf)
    cc = cbuf[...]
    colp = _it() & 7

    def zinit(i, a):
        row, col = _pair(i)
        plsc.store_scatter(zbuf, [row, col], jnp.zeros((16,), F32))
        return a
    lax.fori_loop(0, NPT // 2, zinit, 0)

    def oinit(i, a):
        row, col = _pair(i)
        plsc.store_scatter(obuf, [row, col], jnp.ones((16,), F32))
        return a
    lax.fori_loop(0, W // 2, oinit, 0)

    n0 = sid * NPT
    pltpu.sync_copy(zbuf, ns_sh.at[pl.ds(n0, NPT)])
    pltpu.sync_copy(zbuf, deg_sh.at[pl.ds(n0, NPT)])
    plsc.subcore_barrier()

    e0 = wid * EPT

    def win(w, a):
        pltpu.sync_copy(dst3.at[pl.ds(e0 + w * W, W)], didx)
        pltpu.sync_copy(msg.at[pl.ds(e0 + w * W, W)], mbuf)

        def pair(i, b):
            row, col = _pair(i)
            m2 = plsc.load_gather(mbuf, [row, col])
            plsc.store_scatter(lbuf, [row, col], _log1p(cc * m2))
            return b
        lax.fori_loop(0, W // 2, pair, 0)
        pltpu.sync_copy(lbuf, ns_sh.at[didx], add=True)
        pltpu.sync_copy(obuf, deg_sh.at[didx], add=True)
        return a
    lax.fori_loop(0, NWIN, win, 0)

    plsc.subcore_barrier()
    pltpu.sync_copy(ns_sh.at[pl.ds(n0, NPT)], ns_out.at[cid].at[pl.ds(n0, NPT)])
    pltpu.sync_copy(deg_sh.at[pl.ds(n0, NPT)], deg_out.at[cid].at[pl.ds(n0, NPT)])


def _gs_body(msg, nsp, degp, src3, dst3, cvec, bovn, newmsg, nsp_out,
             sidx, didx, ebuf, mbuf, obuf, lbuf,
             nsA, nsB, dgA, dgB, pbuf, htmp, hall, cbuf, bbuf,
             ns_sh, e_sh, h_sh):
    cid = lax.axis_index("c")
    sid = lax.axis_index("s")
    wid = cid * NS + sid
    pltpu.sync_copy(cvec, cbuf)
    pltpu.sync_copy(bovn, bbuf)
    cc = cbuf[...]
    it = _it()

    n0 = sid * NPT
    pltpu.sync_copy(nsp.at[0].at[pl.ds(n0, NPT)], nsA)
    pltpu.sync_copy(nsp.at[1].at[pl.ds(n0, NPT)], nsB)
    pltpu.sync_copy(degp.at[0].at[pl.ds(n0, NPT)], dgA)
    pltpu.sync_copy(degp.at[1].at[pl.ds(n0, NPT)], dgB)

    def hstep(i, hacc):
        row, col = _pair(i)
        a = plsc.load_gather(nsA, [row, col]) + plsc.load_gather(nsB, [row, col])
        dv = plsc.load_gather(dgA, [row, col]) + plsc.load_gather(dgB, [row, col])
        p = jnp.exp(a - _hmax(a))
        psi = p / _hsum(p)
        plsc.store_scatter(pbuf, [row, col], p)
        plsc.store_scatter(nsB, [row, col], jnp.zeros((16,), F32))
        return hacc + dv * psi
    hacc = lax.fori_loop(0, NPT // 2, hstep, jnp.zeros((16,), F32))
    htmp[...] = hacc
    pltpu.sync_copy(htmp, h_sh.at[pl.ds(sid * 16, 16)])
    plsc.subcore_barrier()
    pltpu.sync_copy(h_sh, hall)
    hs = jnp.zeros((16,), F32)
    for k in range(NS):
        hs = hs + hall[pl.ds(k * 16, 16)]
    hs = hs + _perm(hs, it ^ 8)
    ehv = jnp.exp(-bbuf[...] * hs)

    def estep(i, a):
        row, col = _pair(i)
        p = plsc.load_gather(pbuf, [row, col])
        plsc.store_scatter(pbuf, [row, col], p * ehv)
        return a
    lax.fori_loop(0, NPT // 2, estep, 0)
    pltpu.sync_copy(pbuf, e_sh.at[pl.ds(n0, NPT)])
    pltpu.sync_copy(nsB, ns_sh.at[pl.ds(n0, NPT)])
    plsc.subcore_barrier()

    e0 = wid * EPT
    rev0 = jnp.where(wid < NS, e0 + EH, e0 - EH)

    def win(w, a):
        pltpu.sync_copy(src3.at[pl.ds(e0 + w * W, W)], sidx)
        pltpu.sync_copy(dst3.at[pl.ds(e0 + w * W, W)], didx)
        pltpu.sync_copy(msg.at[pl.ds(rev0 + w * W, W)], mbuf)
        pltpu.sync_copy(e_sh.at[sidx], ebuf)

        def pair(i, b):
            row, col = _pair(i)
            ev = plsc.load_gather(ebuf, [row, col])
            mv = plsc.load_gather(mbuf, [row, col])
            wv = ev / (cc * mv + 1.0)
            nm = wv / _hsum(wv)
            plsc.store_scatter(obuf, [row, col], nm)
            plsc.store_scatter(lbuf, [row, col], _log1p(cc * nm))
            return b
        lax.fori_loop(0, W // 2, pair, 0)
        pltpu.sync_copy(obuf, newmsg.at[pl.ds(e0 + w * W, W)])
        pltpu.sync_copy(lbuf, ns_sh.at[didx], add=True)
        return a
    lax.fori_loop(0, NWIN, win, 0)

    plsc.subcore_barrier()
    pltpu.sync_copy(ns_sh.at[pl.ds(n0, NPT)], nsp_out.at[cid].at[pl.ds(n0, NPT)])


def _marg_body(nsp, degp, bovn, psi_out,
               nsA, nsB, dgA, dgB, obuf, htmp, hall, bbuf, h_sh):
    cid = lax.axis_index("c")
    sid = lax.axis_index("s")
    wid = cid * NS + sid
    pltpu.sync_copy(bovn, bbuf)
    it = _it()

    n0 = sid * NPT
    pltpu.sync_copy(nsp.at[0].at[pl.ds(n0, NPT)], nsA)
    pltpu.sync_copy(nsp.at[1].at[pl.ds(n0, NPT)], nsB)
    pltpu.sync_copy(degp.at[0].at[pl.ds(n0, NPT)], dgA)
    pltpu.sync_copy(degp.at[1].at[pl.ds(n0, NPT)], dgB)

    def hstep(i, hacc):
        row, col = _pair(i)
        a = plsc.load_gather(nsA, [row, col]) + plsc.load_gather(nsB, [row, col])
        dv = plsc.load_gather(dgA, [row, col]) + plsc.load_gather(dgB, [row, col])
        p = jnp.exp(a - _hmax(a))
        psi = p / _hsum(p)
        return hacc + dv * psi
    hacc = lax.fori_loop(0, NPT // 2, hstep, jnp.zeros((16,), F32))
    htmp[...] = hacc
    pltpu.sync_copy(htmp, h_sh.at[pl.ds(sid * 16, 16)])
    plsc.subcore_barrier()
    pltpu.sync_copy(h_sh, hall)
    hs = jnp.zeros((16,), F32)
    for k in range(NS):
        hs = hs + hall[pl.ds(k * 16, 16)]
    hs = hs + _perm(hs, it ^ 8)
    hv = bbuf[...] * hs

    g0 = wid * NPO
    pltpu.sync_copy(nsp.at[0].at[pl.ds(g0, NPO)], nsA.at[pl.ds(0, NPO)])
    pltpu.sync_copy(nsp.at[1].at[pl.ds(g0, NPO)], nsB.at[pl.ds(0, NPO)])

    def ostep(i, a):
        row, col = _pair(i)
        v = (plsc.load_gather(nsA, [row, col])
             + plsc.load_gather(nsB, [row, col]) - hv)
        p = jnp.exp(v - _hmax(v))
        psi = p / _hsum(p)
        plsc.store_scatter(obuf, [row, col], psi)
        return a
    lax.fori_loop(0, NPO // 2, ostep, 0)
    pltpu.sync_copy(obuf, psi_out.at[pl.ds(g0, NPO)])


def _mesh():
    return plsc.VectorSubcoreMesh(core_axis_name="c", subcore_axis_name="s")


_CPARAMS = pltpu.CompilerParams(needs_layout_passes=False, use_tc_tiling_on_sc=False)


@jax.jit
def _run(msg0, src3, dst3, cvec, bovn):
    k0 = pl.kernel(
        _k0_body,
        out_type=(jax.ShapeDtypeStruct((NC, NP, Q), F32),
                  jax.ShapeDtypeStruct((NC, NP, Q), F32)),
        mesh=_mesh(),
        compiler_params=_CPARAMS,
        scratch_types=[
            pltpu.VMEM((W,), I32),
            pltpu.VMEM((W, Q), F32),
            pltpu.VMEM((W, Q), F32),
            pltpu.VMEM((W, Q), F32),
            pltpu.VMEM((NPT, Q), F32),
            pltpu.VMEM((16,), F32),
            pltpu.VMEM_SHARED((NP, Q), F32),
            pltpu.VMEM_SHARED((NP, Q), F32),
        ],
    )
    gs = pl.kernel(
        _gs_body,
        out_type=(jax.ShapeDtypeStruct((E2, Q), F32),
                  jax.ShapeDtypeStruct((NC, NP, Q), F32)),
        mesh=_mesh(),
        compiler_params=_CPARAMS,
        scratch_types=[
            pltpu.VMEM((W,), I32),
            pltpu.VMEM((W,), I32),
            pltpu.VMEM((W, Q), F32),
            pltpu.VMEM((W, Q), F32),
            pltpu.VMEM((W, Q), F32),
            pltpu.VMEM((W, Q), F32),
            pltpu.VMEM((NPT, Q), F32),
            pltpu.VMEM((NPT, Q), F32),
            pltpu.VMEM((NPT, Q), F32),
            pltpu.VMEM((NPT, Q), F32),
            pltpu.VMEM((NPT, Q), F32),
            pltpu.VMEM((16,), F32),
            pltpu.VMEM((NS * 16,), F32),
            pltpu.VMEM((16,), F32),
            pltpu.VMEM((16,), F32),
            pltpu.VMEM_SHARED((NP, Q), F32),
            pltpu.VMEM_SHARED((NP, Q), F32),
            pltpu.VMEM_SHARED((NS * 16,), F32),
        ],
    )
    marg = pl.kernel(
        _marg_body,
        out_type=jax.ShapeDtypeStruct((NP, Q), F32),
        mesh=_mesh(),
        compiler_params=_CPARAMS,
        scratch_types=[
            pltpu.VMEM((NPT, Q), F32),
            pltpu.VMEM((NPT, Q), F32),
            pltpu.VMEM((NPT, Q), F32),
            pltpu.VMEM((NPT, Q), F32),
            pltpu.VMEM((NPO, Q), F32),
            pltpu.VMEM((16,), F32),
            pltpu.VMEM((NS * 16,), F32),
            pltpu.VMEM((16,), F32),
            pltpu.VMEM_SHARED((NS * 16,), F32),
        ],
    )
    nsp, degp = k0(msg0, dst3, cvec)
    msg = msg0
    for _ in range(5):
        msg, nsp = gs(msg, nsp, degp, src3, dst3, cvec, bovn)
    psi_pad = marg(nsp, degp, bovn)
    return msg, psi_pad[:N_NODES]


def kernel(edge_index, num_nodes, beta, message_map_init):
    src = jnp.concatenate([edge_index[0], edge_index[1]]).astype(I32)
    dst = jnp.concatenate([edge_index[1], edge_index[0]]).astype(I32)
    src3 = src
    dst3 = dst
    beta = jnp.asarray(beta, F32)
    cvec = jnp.full((16,), jnp.exp(beta) - 1.0, F32)
    bovn = jnp.full((16,), beta / jnp.asarray(num_nodes, F32), F32)
    return _run(message_map_init, src3, dst3, cvec, bovn)


# speedup vs baseline: 20.2218x; 2.6739x over previous
import jax
import jax.numpy as jnp
from jax import lax
from jax.experimental import pallas as pl
from jax.experimental.pallas import tpu as pltpu
from jax.experimental.pallas import tpu_sc as plsc

N_NODES = 10000
NP = 10240
E2 = 640000
EH = 320000
Q = 8
NC, NS, NW = 2, 16, 32
EPT = E2 // NW
W = 1000
NWIN = EPT // W
CH = 125
NCH = W // CH
NPT = NP // NS
NPO = NP // NW

F32 = jnp.float32
I32 = jnp.int32

_LOG1P_COEF = (0.9999947246324665, -0.49984485381321225, 0.3317124871811307,
               -0.2413461885986923, 0.17221571318743972, -0.10745355469223546,
               0.052375598963768255, -0.01786311035930909, 0.0037034338062475664,
               -0.0003470312864672028)


def _it():
    return lax.iota(I32, 16)


def _perm(v, p):
    dnums = lax.GatherDimensionNumbers(
        offset_dims=(), collapsed_slice_dims=(0,), start_index_map=(0,))
    return lax.gather(v, p[:, None], dnums, (1,),
                      mode=lax.GatherScatterMode.PROMISE_IN_BOUNDS)


def _hmax(v):
    it = _it()
    for k in (1, 2, 4):
        v = jnp.maximum(v, _perm(v, it ^ k))
    return v


def _hsum(v):
    it = _it()
    for k in (1, 2, 4):
        v = v + _perm(v, it ^ k)
    return v


def _log1p(u):
    q = u * _LOG1P_COEF[9]
    for k in range(8, 0, -1):
        q = (q + _LOG1P_COEF[k]) * u
    return (q + _LOG1P_COEF[0]) * u


def _pair(i):
    it = _it()
    return 2 * i + (it >> 3), it & 7


def _k0_body(msg, dst3, cvec, ns_out, deg_out,
             didx, mbuf, lbuf, obuf, zbuf, cbuf, ns_sh, deg_sh):
    cid = lax.axis_index("c")
    sid = lax.axis_index("s")
    wid = cid * NS + sid
    pltpu.sync_copy(cvec, cbuf)
    cc = cbuf[...]
    colp = _it() & 7

    @plsc.parallel_loop(0, NPT // 2, unroll=8)
    def zinit(i):
        row, col = _pair(i)
        plsc.store_scatter(zbuf, [row, col], jnp.zeros((16,), F32))

    @plsc.parallel_loop(0, W // 2, unroll=8)
    def oinit(i):
        row, col = _pair(i)
        plsc.store_scatter(obuf, [row, col], jnp.ones((16,), F32))

    n0 = sid * NPT
    pltpu.sync_copy(zbuf, ns_sh.at[pl.ds(n0, NPT)])
    pltpu.sync_copy(zbuf, deg_sh.at[pl.ds(n0, NPT)])
    plsc.subcore_barrier()

    e0 = wid * EPT

    def win(w, a):
        pltpu.sync_copy(dst3.at[pl.ds(e0 + w * W, W)], didx)
        pltpu.sync_copy(msg.at[pl.ds(e0 + w * W, W)], mbuf)

        @plsc.parallel_loop(0, W // 2, unroll=8)
        def pair(i):
            row, col = _pair(i)
            m2 = plsc.load_gather(mbuf, [row, col])
            plsc.store_scatter(lbuf, [row, col], _log1p(cc * m2))
        pltpu.sync_copy(lbuf, ns_sh.at[didx], add=True)
        pltpu.sync_copy(obuf, deg_sh.at[didx], add=True)
        return a
    lax.fori_loop(0, NWIN, win, 0)

    plsc.subcore_barrier()
    pltpu.sync_copy(ns_sh.at[pl.ds(n0, NPT)], ns_out.at[cid].at[pl.ds(n0, NPT)])
    pltpu.sync_copy(deg_sh.at[pl.ds(n0, NPT)], deg_out.at[cid].at[pl.ds(n0, NPT)])


def _gs_body(msg, nsp, degp, src3, dst3, cvec, bovn, newmsg, nsp_out,
             sidx, didx, ebuf, mbuf, obuf, lbuf,
             nsA, nsB, dgA, dgB, pbuf, htmp, hall, cbuf, bbuf,
             ns_sh, e_sh, h_sh):
    cid = lax.axis_index("c")
    sid = lax.axis_index("s")
    wid = cid * NS + sid
    pltpu.sync_copy(cvec, cbuf)
    pltpu.sync_copy(bovn, bbuf)
    cc = cbuf[...]
    it = _it()

    n0 = sid * NPT
    pltpu.sync_copy(nsp.at[0].at[pl.ds(n0, NPT)], nsA)
    pltpu.sync_copy(nsp.at[1].at[pl.ds(n0, NPT)], nsB)
    pltpu.sync_copy(degp.at[0].at[pl.ds(n0, NPT)], dgA)
    pltpu.sync_copy(degp.at[1].at[pl.ds(n0, NPT)], dgB)

    def hstep(i, hacc):
        row, col = _pair(i)
        a = plsc.load_gather(nsA, [row, col]) + plsc.load_gather(nsB, [row, col])
        dv = plsc.load_gather(dgA, [row, col]) + plsc.load_gather(dgB, [row, col])
        p = jnp.exp(a - _hmax(a))
        psi = p / _hsum(p)
        plsc.store_scatter(pbuf, [row, col], p)
        plsc.store_scatter(nsB, [row, col], jnp.zeros((16,), F32))
        return hacc + dv * psi
    hacc = plsc.parallel_loop(0, NPT // 2, unroll=4,
                              carry=jnp.zeros((16,), F32))(hstep)
    htmp[...] = hacc
    pltpu.sync_copy(htmp, h_sh.at[pl.ds(sid * 16, 16)])
    plsc.subcore_barrier()
    pltpu.sync_copy(h_sh, hall)
    hs = jnp.zeros((16,), F32)
    for k in range(NS):
        hs = hs + hall[pl.ds(k * 16, 16)]
    hs = hs + _perm(hs, it ^ 8)
    ehv = jnp.exp(-bbuf[...] * hs)

    @plsc.parallel_loop(0, NPT // 2, unroll=8)
    def estep(i):
        row, col = _pair(i)
        p = plsc.load_gather(pbuf, [row, col])
        plsc.store_scatter(pbuf, [row, col], p * ehv)
    pltpu.sync_copy(pbuf, e_sh.at[pl.ds(n0, NPT)])
    pltpu.sync_copy(nsB, ns_sh.at[pl.ds(n0, NPT)])
    plsc.subcore_barrier()

    e0 = wid * EPT
    rev0 = jnp.where(wid < NS, e0 + EH, e0 - EH)

    def win(w, a):
        pltpu.sync_copy(src3.at[pl.ds(e0 + w * W, W)], sidx)
        pltpu.sync_copy(dst3.at[pl.ds(e0 + w * W, W)], didx)
        pltpu.sync_copy(msg.at[pl.ds(rev0 + w * W, W)], mbuf)
        pltpu.sync_copy(e_sh.at[sidx], ebuf)

        @plsc.parallel_loop(0, W // 2, unroll=8)
        def pair(i):
            row, col = _pair(i)
            ev = plsc.load_gather(ebuf, [row, col])
            mv = plsc.load_gather(mbuf, [row, col])
            wv = ev / (cc * mv + 1.0)
            nm = wv / _hsum(wv)
            plsc.store_scatter(obuf, [row, col], nm)
            plsc.store_scatter(lbuf, [row, col], _log1p(cc * nm))
        pltpu.sync_copy(obuf, newmsg.at[pl.ds(e0 + w * W, W)])
        pltpu.sync_copy(lbuf, ns_sh.at[didx], add=True)
        return a
    lax.fori_loop(0, NWIN, win, 0)

    plsc.subcore_barrier()
    pltpu.sync_copy(ns_sh.at[pl.ds(n0, NPT)], nsp_out.at[cid].at[pl.ds(n0, NPT)])


def _marg_body(nsp, degp, bovn, psi_out,
               nsA, nsB, dgA, dgB, obuf, htmp, hall, bbuf, h_sh):
    cid = lax.axis_index("c")
    sid = lax.axis_index("s")
    wid = cid * NS + sid
    pltpu.sync_copy(bovn, bbuf)
    it = _it()

    n0 = sid * NPT
    pltpu.sync_copy(nsp.at[0].at[pl.ds(n0, NPT)], nsA)
    pltpu.sync_copy(nsp.at[1].at[pl.ds(n0, NPT)], nsB)
    pltpu.sync_copy(degp.at[0].at[pl.ds(n0, NPT)], dgA)
    pltpu.sync_copy(degp.at[1].at[pl.ds(n0, NPT)], dgB)

    def hstep(i, hacc):
        row, col = _pair(i)
        a = plsc.load_gather(nsA, [row, col]) + plsc.load_gather(nsB, [row, col])
        dv = plsc.load_gather(dgA, [row, col]) + plsc.load_gather(dgB, [row, col])
        p = jnp.exp(a - _hmax(a))
        psi = p / _hsum(p)
        return hacc + dv * psi
    hacc = plsc.parallel_loop(0, NPT // 2, unroll=4,
                              carry=jnp.zeros((16,), F32))(hstep)
    htmp[...] = hacc
    pltpu.sync_copy(htmp, h_sh.at[pl.ds(sid * 16, 16)])
    plsc.subcore_barrier()
    pltpu.sync_copy(h_sh, hall)
    hs = jnp.zeros((16,), F32)
    for k in range(NS):
        hs = hs + hall[pl.ds(k * 16, 16)]
    hs = hs + _perm(hs, it ^ 8)
    hv = bbuf[...] * hs

    g0 = wid * NPO
    pltpu.sync_copy(nsp.at[0].at[pl.ds(g0, NPO)], nsA.at[pl.ds(0, NPO)])
    pltpu.sync_copy(nsp.at[1].at[pl.ds(g0, NPO)], nsB.at[pl.ds(0, NPO)])

    @plsc.parallel_loop(0, NPO // 2, unroll=4)
    def ostep(i):
        row, col = _pair(i)
        v = (plsc.load_gather(nsA, [row, col])
             + plsc.load_gather(nsB, [row, col]) - hv)
        p = jnp.exp(v - _hmax(v))
        psi = p / _hsum(p)
        plsc.store_scatter(obuf, [row, col], psi)
    pltpu.sync_copy(obuf, psi_out.at[pl.ds(g0, NPO)])


def _mesh():
    return plsc.VectorSubcoreMesh(core_axis_name="c", subcore_axis_name="s")


_CPARAMS = pltpu.CompilerParams(needs_layout_passes=False, use_tc_tiling_on_sc=False)


@jax.jit
def _run(msg0, src3, dst3, cvec, bovn):
    k0 = pl.kernel(
        _k0_body,
        out_type=(jax.ShapeDtypeStruct((NC, NP, Q), F32),
                  jax.ShapeDtypeStruct((NC, NP, Q), F32)),
        mesh=_mesh(),
        compiler_params=_CPARAMS,
        scratch_types=[
            pltpu.VMEM((W,), I32),
            pltpu.VMEM((W, Q), F32),
            pltpu.VMEM((W, Q), F32),
            pltpu.VMEM((W, Q), F32),
            pltpu.VMEM((NPT, Q), F32),
            pltpu.VMEM((16,), F32),
            pltpu.VMEM_SHARED((NP, Q), F32),
            pltpu.VMEM_SHARED((NP, Q), F32),
        ],
    )
    gs = pl.kernel(
        _gs_body,
        out_type=(jax.ShapeDtypeStruct((E2, Q), F32),
                  jax.ShapeDtypeStruct((NC, NP, Q), F32)),
        mesh=_mesh(),
        compiler_params=_CPARAMS,
        scratch_types=[
            pltpu.VMEM((W,), I32),
            pltpu.VMEM((W,), I32),
            pltpu.VMEM((W, Q), F32),
            pltpu.VMEM((W, Q), F32),
            pltpu.VMEM((W, Q), F32),
            pltpu.VMEM((W, Q), F32),
            pltpu.VMEM((NPT, Q), F32),
            pltpu.VMEM((NPT, Q), F32),
            pltpu.VMEM((NPT, Q), F32),
            pltpu.VMEM((NPT, Q), F32),
            pltpu.VMEM((NPT, Q), F32),
            pltpu.VMEM((16,), F32),
            pltpu.VMEM((NS * 16,), F32),
            pltpu.VMEM((16,), F32),
            pltpu.VMEM((16,), F32),
            pltpu.VMEM_SHARED((NP, Q), F32),
            pltpu.VMEM_SHARED((NP, Q), F32),
            pltpu.VMEM_SHARED((NS * 16,), F32),
        ],
    )
    marg = pl.kernel(
        _marg_body,
        out_type=jax.ShapeDtypeStruct((NP, Q), F32),
        mesh=_mesh(),
        compiler_params=_CPARAMS,
        scratch_types=[
            pltpu.VMEM((NPT, Q), F32),
            pltpu.VMEM((NPT, Q), F32),
            pltpu.VMEM((NPT, Q), F32),
            pltpu.VMEM((NPT, Q), F32),
            pltpu.VMEM((NPO, Q), F32),
            pltpu.VMEM((16,), F32),
            pltpu.VMEM((NS * 16,), F32),
            pltpu.VMEM((16,), F32),
            pltpu.VMEM_SHARED((NS * 16,), F32),
        ],
    )
    nsp, degp = k0(msg0, dst3, cvec)
    msg = msg0
    for _ in range(5):
        msg, nsp = gs(msg, nsp, degp, src3, dst3, cvec, bovn)
    psi_pad = marg(nsp, degp, bovn)
    return msg, psi_pad[:N_NODES]


def kernel(edge_index, num_nodes, beta, message_map_init):
    src = jnp.concatenate([edge_index[0], edge_index[1]]).astype(I32)
    dst = jnp.concatenate([edge_index[1], edge_index[0]]).astype(I32)
    src3 = src
    dst3 = dst
    beta = jnp.asarray(beta, F32)
    cvec = jnp.full((16,), jnp.exp(beta) - 1.0, F32)
    bovn = jnp.full((16,), beta / jnp.asarray(num_nodes, F32), F32)
    return _run(message_map_init, src3, dst3, cvec, bovn)


# speedup vs baseline: 20.2458x; 1.0012x over previous
import jax
import jax.numpy as jnp
from jax import lax
from jax.experimental import pallas as pl
from jax.experimental.pallas import tpu as pltpu
from jax.experimental.pallas import tpu_sc as plsc

N_NODES = 10000
NP = 10240
E2 = 640000
EH = 320000
Q = 8
NC, NS, NW = 2, 16, 32
EPT = E2 // NW
W = 1000
NWIN = EPT // W
CH = 125
NCH = W // CH
NPT = NP // NS
NPO = NP // NW

F32 = jnp.float32
I32 = jnp.int32

_LOG1P_COEF = (0.9999947246324665, -0.49984485381321225, 0.3317124871811307,
               -0.2413461885986923, 0.17221571318743972, -0.10745355469223546,
               0.052375598963768255, -0.01786311035930909, 0.0037034338062475664,
               -0.0003470312864672028)


def _it():
    return lax.iota(I32, 16)


def _perm(v, p):
    dnums = lax.GatherDimensionNumbers(
        offset_dims=(), collapsed_slice_dims=(0,), start_index_map=(0,))
    return lax.gather(v, p[:, None], dnums, (1,),
                      mode=lax.GatherScatterMode.PROMISE_IN_BOUNDS)


def _hmax(v):
    it = _it()
    for k in (1, 2, 4):
        v = jnp.maximum(v, _perm(v, it ^ k))
    return v


def _hsum(v):
    it = _it()
    for k in (1, 2, 4):
        v = v + _perm(v, it ^ k)
    return v


def _log1p(u):
    q = u * _LOG1P_COEF[9]
    for k in range(8, 0, -1):
        q = (q + _LOG1P_COEF[k]) * u
    return (q + _LOG1P_COEF[0]) * u


def _pair(i):
    it = _it()
    return 2 * i + (it >> 3), it & 7


def _k0_body(msg, ei, cvec, ns_out, deg_out,
             didx, mbuf, lbuf, obuf, zbuf, cbuf, ns_sh, deg_sh):
    cid = lax.axis_index("c")
    sid = lax.axis_index("s")
    wid = cid * NS + sid
    pltpu.sync_copy(cvec, cbuf)
    cc = cbuf[...]
    colp = _it() & 7

    @plsc.parallel_loop(0, NPT // 2, unroll=8)
    def zinit(i):
        row, col = _pair(i)
        plsc.store_scatter(zbuf, [row, col], jnp.zeros((16,), F32))

    @plsc.parallel_loop(0, W // 2, unroll=8)
    def oinit(i):
        row, col = _pair(i)
        plsc.store_scatter(obuf, [row, col], jnp.ones((16,), F32))

    n0 = sid * NPT
    pltpu.sync_copy(zbuf, ns_sh.at[pl.ds(n0, NPT)])
    pltpu.sync_copy(zbuf, deg_sh.at[pl.ds(n0, NPT)])
    plsc.subcore_barrier()

    e0 = wid * EPT
    half = wid // NS
    eo = e0 - half * EH

    def win(w, a):
        pltpu.sync_copy(ei.at[1 - half].at[pl.ds(eo + w * W, W)], didx)
        pltpu.sync_copy(msg.at[pl.ds(e0 + w * W, W)], mbuf)

        @plsc.parallel_loop(0, W // 2, unroll=8)
        def pair(i):
            row, col = _pair(i)
            m2 = plsc.load_gather(mbuf, [row, col])
            plsc.store_scatter(lbuf, [row, col], _log1p(cc * m2))
        pltpu.sync_copy(lbuf, ns_sh.at[didx], add=True)
        pltpu.sync_copy(obuf, deg_sh.at[didx], add=True)
        return a
    lax.fori_loop(0, NWIN, win, 0)

    plsc.subcore_barrier()
    pltpu.sync_copy(ns_sh.at[pl.ds(n0, NPT)], ns_out.at[cid].at[pl.ds(n0, NPT)])
    pltpu.sync_copy(deg_sh.at[pl.ds(n0, NPT)], deg_out.at[cid].at[pl.ds(n0, NPT)])


def _gs_body(msg, nsp, degp, ei, cvec, bovn, newmsg, nsp_out,
             sidx, didx, ebuf, mbuf, obuf, lbuf,
             nsA, nsB, dgA, dgB, pbuf, htmp, hall, cbuf, bbuf,
             ns_sh, e_sh, h_sh):
    cid = lax.axis_index("c")
    sid = lax.axis_index("s")
    wid = cid * NS + sid
    pltpu.sync_copy(cvec, cbuf)
    pltpu.sync_copy(bovn, bbuf)
    cc = cbuf[...]
    it = _it()

    n0 = sid * NPT
    pltpu.sync_copy(nsp.at[0].at[pl.ds(n0, NPT)], nsA)
    pltpu.sync_copy(nsp.at[1].at[pl.ds(n0, NPT)], nsB)
    pltpu.sync_copy(degp.at[0].at[pl.ds(n0, NPT)], dgA)
    pltpu.sync_copy(degp.at[1].at[pl.ds(n0, NPT)], dgB)

    def hstep(i, hacc):
        row, col = _pair(i)
        a = plsc.load_gather(nsA, [row, col]) + plsc.load_gather(nsB, [row, col])
        dv = plsc.load_gather(dgA, [row, col]) + plsc.load_gather(dgB, [row, col])
        p = jnp.exp(a - _hmax(a))
        psi = p / _hsum(p)
        plsc.store_scatter(pbuf, [row, col], p)
        plsc.store_scatter(nsB, [row, col], jnp.zeros((16,), F32))
        return hacc + dv * psi
    hacc = plsc.parallel_loop(0, NPT // 2, unroll=4,
                              carry=jnp.zeros((16,), F32))(hstep)
    htmp[...] = hacc
    pltpu.sync_copy(htmp, h_sh.at[pl.ds(sid * 16, 16)])
    plsc.subcore_barrier()
    pltpu.sync_copy(h_sh, hall)
    hs = jnp.zeros((16,), F32)
    for k in range(NS):
        hs = hs + hall[pl.ds(k * 16, 16)]
    hs = hs + _perm(hs, it ^ 8)
    ehv = jnp.exp(-bbuf[...] * hs)

    @plsc.parallel_loop(0, NPT // 2, unroll=8)
    def estep(i):
        row, col = _pair(i)
        p = plsc.load_gather(pbuf, [row, col])
        plsc.store_scatter(pbuf, [row, col], p * ehv)
    pltpu.sync_copy(pbuf, e_sh.at[pl.ds(n0, NPT)])
    pltpu.sync_copy(nsB, ns_sh.at[pl.ds(n0, NPT)])
    plsc.subcore_barrier()

    e0 = wid * EPT
    half = wid // NS
    eo = e0 - half * EH
    rev0 = jnp.where(wid < NS, e0 + EH, e0 - EH)

    def win(w, a):
        pltpu.sync_copy(ei.at[half].at[pl.ds(eo + w * W, W)], sidx)
        pltpu.sync_copy(ei.at[1 - half].at[pl.ds(eo + w * W, W)], didx)
        pltpu.sync_copy(msg.at[pl.ds(rev0 + w * W, W)], mbuf)
        pltpu.sync_copy(e_sh.at[sidx], ebuf)

        @plsc.parallel_loop(0, W // 2, unroll=8)
        def pair(i):
            row, col = _pair(i)
            ev = plsc.load_gather(ebuf, [row, col])
            mv = plsc.load_gather(mbuf, [row, col])
            wv = ev / (cc * mv + 1.0)
            nm = wv / _hsum(wv)
            plsc.store_scatter(obuf, [row, col], nm)
            plsc.store_scatter(lbuf, [row, col], _log1p(cc * nm))
        pltpu.sync_copy(obuf, newmsg.at[pl.ds(e0 + w * W, W)])
        pltpu.sync_copy(lbuf, ns_sh.at[didx], add=True)
        return a
    lax.fori_loop(0, NWIN, win, 0)

    plsc.subcore_barrier()
    pltpu.sync_copy(ns_sh.at[pl.ds(n0, NPT)], nsp_out.at[cid].at[pl.ds(n0, NPT)])


def _marg_body(nsp, degp, bovn, psi_out,
               nsA, nsB, dgA, dgB, obuf, htmp, hall, bbuf, h_sh):
    cid = lax.axis_index("c")
    sid = lax.axis_index("s")
    wid = cid * NS + sid
    pltpu.sync_copy(bovn, bbuf)
    it = _it()

    n0 = sid * NPT
    pltpu.sync_copy(nsp.at[0].at[pl.ds(n0, NPT)], nsA)
    pltpu.sync_copy(nsp.at[1].at[pl.ds(n0, NPT)], nsB)
    pltpu.sync_copy(degp.at[0].at[pl.ds(n0, NPT)], dgA)
    pltpu.sync_copy(degp.at[1].at[pl.ds(n0, NPT)], dgB)

    def hstep(i, hacc):
        row, col = _pair(i)
        a = plsc.load_gather(nsA, [row, col]) + plsc.load_gather(nsB, [row, col])
        dv = plsc.load_gather(dgA, [row, col]) + plsc.load_gather(dgB, [row, col])
        p = jnp.exp(a - _hmax(a))
        psi = p / _hsum(p)
        return hacc + dv * psi
    hacc = plsc.parallel_loop(0, NPT // 2, unroll=4,
                              carry=jnp.zeros((16,), F32))(hstep)
    htmp[...] = hacc
    pltpu.sync_copy(htmp, h_sh.at[pl.ds(sid * 16, 16)])
    plsc.subcore_barrier()
    pltpu.sync_copy(h_sh, hall)
    hs = jnp.zeros((16,), F32)
    for k in range(NS):
        hs = hs + hall[pl.ds(k * 16, 16)]
    hs = hs + _perm(hs, it ^ 8)
    hv = bbuf[...] * hs

    g0 = wid * NPO
    pltpu.sync_copy(nsp.at[0].at[pl.ds(g0, NPO)], nsA.at[pl.ds(0, NPO)])
    pltpu.sync_copy(nsp.at[1].at[pl.ds(g0, NPO)], nsB.at[pl.ds(0, NPO)])

    @plsc.parallel_loop(0, NPO // 2, unroll=4)
    def ostep(i):
        row, col = _pair(i)
        v = (plsc.load_gather(nsA, [row, col])
             + plsc.load_gather(nsB, [row, col]) - hv)
        p = jnp.exp(v - _hmax(v))
        psi = p / _hsum(p)
        plsc.store_scatter(obuf, [row, col], psi)
    pltpu.sync_copy(obuf, psi_out.at[pl.ds(g0, NPO)])


def _mesh():
    return plsc.VectorSubcoreMesh(core_axis_name="c", subcore_axis_name="s")


_CPARAMS = pltpu.CompilerParams(needs_layout_passes=False, use_tc_tiling_on_sc=False)


@jax.jit
def _run(msg0, ei, cvec, bovn):
    k0 = pl.kernel(
        _k0_body,
        out_type=(jax.ShapeDtypeStruct((NC, NP, Q), F32),
                  jax.ShapeDtypeStruct((NC, NP, Q), F32)),
        mesh=_mesh(),
        compiler_params=_CPARAMS,
        scratch_types=[
            pltpu.VMEM((W,), I32),
            pltpu.VMEM((W, Q), F32),
            pltpu.VMEM((W, Q), F32),
            pltpu.VMEM((W, Q), F32),
            pltpu.VMEM((NPT, Q), F32),
            pltpu.VMEM((16,), F32),
            pltpu.VMEM_SHARED((NP, Q), F32),
            pltpu.VMEM_SHARED((NP, Q), F32),
        ],
    )
    gs = pl.kernel(
        _gs_body,
        out_type=(jax.ShapeDtypeStruct((E2, Q), F32),
                  jax.ShapeDtypeStruct((NC, NP, Q), F32)),
        mesh=_mesh(),
        compiler_params=_CPARAMS,
        scratch_types=[
            pltpu.VMEM((W,), I32),
            pltpu.VMEM((W,), I32),
            pltpu.VMEM((W, Q), F32),
            pltpu.VMEM((W, Q), F32),
            pltpu.VMEM((W, Q), F32),
            pltpu.VMEM((W, Q), F32),
            pltpu.VMEM((NPT, Q), F32),
            pltpu.VMEM((NPT, Q), F32),
            pltpu.VMEM((NPT, Q), F32),
            pltpu.VMEM((NPT, Q), F32),
            pltpu.VMEM((NPT, Q), F32),
            pltpu.VMEM((16,), F32),
            pltpu.VMEM((NS * 16,), F32),
            pltpu.VMEM((16,), F32),
            pltpu.VMEM((16,), F32),
            pltpu.VMEM_SHARED((NP, Q), F32),
            pltpu.VMEM_SHARED((NP, Q), F32),
            pltpu.VMEM_SHARED((NS * 16,), F32),
        ],
    )
    marg = pl.kernel(
        _marg_body,
        out_type=jax.ShapeDtypeStruct((NP, Q), F32),
        mesh=_mesh(),
        compiler_params=_CPARAMS,
        scratch_types=[
            pltpu.VMEM((NPT, Q), F32),
            pltpu.VMEM((NPT, Q), F32),
            pltpu.VMEM((NPT, Q), F32),
            pltpu.VMEM((NPT, Q), F32),
            pltpu.VMEM((NPO, Q), F32),
            pltpu.VMEM((16,), F32),
            pltpu.VMEM((NS * 16,), F32),
            pltpu.VMEM((16,), F32),
            pltpu.VMEM_SHARED((NS * 16,), F32),
        ],
    )
    nsp, degp = k0(msg0, ei, cvec)
    msg = msg0
    for _ in range(5):
        msg, nsp = gs(msg, nsp, degp, ei, cvec, bovn)
    psi_pad = marg(nsp, degp, bovn)
    return msg, psi_pad[:N_NODES]


def kernel(edge_index, num_nodes, beta, message_map_init):
    beta = jnp.asarray(beta, F32)
    cvec = jnp.full((16,), jnp.exp(beta) - 1.0, F32)
    bovn = jnp.full((16,), beta / jnp.asarray(num_nodes, F32), F32)
    return _run(message_map_init, edge_index, cvec, bovn)


# speedup vs baseline: 22.2126x; 1.0971x over previous
import jax
import jax.numpy as jnp
from jax import lax
from jax.experimental import pallas as pl
from jax.experimental.pallas import tpu as pltpu
from jax.experimental.pallas import tpu_sc as plsc

N_NODES = 10000
NP = 10240
E2 = 640000
EH = 320000
Q = 8
NC, NS, NW = 2, 16, 32
EPT = E2 // NW
W = 2000
NWIN = EPT // W
CH = 125
NCH = W // CH
NPT = NP // NS
NPO = NP // NW

F32 = jnp.float32
I32 = jnp.int32

_LOG1P_COEF = (0.9999947246324665, -0.49984485381321225, 0.3317124871811307,
               -0.2413461885986923, 0.17221571318743972, -0.10745355469223546,
               0.052375598963768255, -0.01786311035930909, 0.0037034338062475664,
               -0.0003470312864672028)


def _it():
    return lax.iota(I32, 16)


def _perm(v, p):
    dnums = lax.GatherDimensionNumbers(
        offset_dims=(), collapsed_slice_dims=(0,), start_index_map=(0,))
    return lax.gather(v, p[:, None], dnums, (1,),
                      mode=lax.GatherScatterMode.PROMISE_IN_BOUNDS)


def _hmax(v):
    it = _it()
    for k in (1, 2, 4):
        v = jnp.maximum(v, _perm(v, it ^ k))
    return v


def _hsum(v):
    it = _it()
    for k in (1, 2, 4):
        v = v + _perm(v, it ^ k)
    return v


def _log1p(u):
    q = u * _LOG1P_COEF[9]
    for k in range(8, 0, -1):
        q = (q + _LOG1P_COEF[k]) * u
    return (q + _LOG1P_COEF[0]) * u


def _pair(i):
    it = _it()
    return 2 * i + (it >> 3), it & 7


def _k0_body(msg, dst3, cvec, ns_out, deg_out,
             didx, mbuf, lbuf, obuf, zbuf, cbuf, ns_sh, deg_sh):
    cid = lax.axis_index("c")
    sid = lax.axis_index("s")
    wid = cid * NS + sid
    pltpu.sync_copy(cvec, cbuf)
    cc = cbuf[...]
    colp = _it() & 7

    @plsc.parallel_loop(0, NPT // 2, unroll=8)
    def zinit(i):
        row, col = _pair(i)
        plsc.store_scatter(zbuf, [row, col], jnp.zeros((16,), F32))

    @plsc.parallel_loop(0, W // 2, unroll=8)
    def oinit(i):
        row, col = _pair(i)
        plsc.store_scatter(obuf, [row, col], jnp.ones((16,), F32))

    n0 = sid * NPT
    pltpu.sync_copy(zbuf, ns_sh.at[pl.ds(n0, NPT)])
    pltpu.sync_copy(zbuf, deg_sh.at[pl.ds(n0, NPT)])
    plsc.subcore_barrier()

    e0 = wid * EPT

    def win(w, a):
        pltpu.sync_copy(dst3.at[pl.ds(e0 + w * W, W)], didx)
        pltpu.sync_copy(msg.at[pl.ds(e0 + w * W, W)], mbuf)

        @plsc.parallel_loop(0, W // 2, unroll=8)
        def pair(i):
            row, col = _pair(i)
            m2 = plsc.load_gather(mbuf, [row, col])
            plsc.store_scatter(lbuf, [row, col], _log1p(cc * m2))
        pltpu.sync_copy(lbuf, ns_sh.at[didx], add=True)
        pltpu.sync_copy(obuf, deg_sh.at[didx], add=True)
        return a
    lax.fori_loop(0, NWIN, win, 0)

    plsc.subcore_barrier()
    pltpu.sync_copy(ns_sh.at[pl.ds(n0, NPT)], ns_out.at[cid].at[pl.ds(n0, NPT)])
    pltpu.sync_copy(deg_sh.at[pl.ds(n0, NPT)], deg_out.at[cid].at[pl.ds(n0, NPT)])


def _gs_body(msg, nsp, degp, src3, dst3, cvec, bovn, newmsg, nsp_out,
             sidx, didx, ebuf, mbuf, obuf, lbuf,
             nsA, nsB, dgA, dgB, pbuf, htmp, hall, cbuf, bbuf,
             ns_sh, e_sh, h_sh):
    cid = lax.axis_index("c")
    sid = lax.axis_index("s")
    wid = cid * NS + sid
    pltpu.sync_copy(cvec, cbuf)
    pltpu.sync_copy(bovn, bbuf)
    cc = cbuf[...]
    it = _it()

    n0 = sid * NPT
    pltpu.sync_copy(nsp.at[0].at[pl.ds(n0, NPT)], nsA)
    pltpu.sync_copy(nsp.at[1].at[pl.ds(n0, NPT)], nsB)
    pltpu.sync_copy(degp.at[0].at[pl.ds(n0, NPT)], dgA)
    pltpu.sync_copy(degp.at[1].at[pl.ds(n0, NPT)], dgB)

    def hstep(i, hacc):
        row, col = _pair(i)
        a = plsc.load_gather(nsA, [row, col]) + plsc.load_gather(nsB, [row, col])
        dv = plsc.load_gather(dgA, [row, col]) + plsc.load_gather(dgB, [row, col])
        p = jnp.exp(a - _hmax(a))
        psi = p / _hsum(p)
        plsc.store_scatter(pbuf, [row, col], p)
        plsc.store_scatter(nsB, [row, col], jnp.zeros((16,), F32))
        return hacc + dv * psi
    hacc = plsc.parallel_loop(0, NPT // 2, unroll=4,
                              carry=jnp.zeros((16,), F32))(hstep)
    htmp[...] = hacc
    pltpu.sync_copy(htmp, h_sh.at[pl.ds(sid * 16, 16)])
    plsc.subcore_barrier()
    pltpu.sync_copy(h_sh, hall)
    hs = jnp.zeros((16,), F32)
    for k in range(NS):
        hs = hs + hall[pl.ds(k * 16, 16)]
    hs = hs + _perm(hs, it ^ 8)
    ehv = jnp.exp(-bbuf[...] * hs)

    @plsc.parallel_loop(0, NPT // 2, unroll=8)
    def estep(i):
        row, col = _pair(i)
        p = plsc.load_gather(pbuf, [row, col])
        plsc.store_scatter(pbuf, [row, col], p * ehv)
    pltpu.sync_copy(pbuf, e_sh.at[pl.ds(n0, NPT)])
    pltpu.sync_copy(nsB, ns_sh.at[pl.ds(n0, NPT)])
    plsc.subcore_barrier()

    e0 = wid * EPT
    rev0 = jnp.where(wid < NS, e0 + EH, e0 - EH)

    def win(w, a):
        pltpu.sync_copy(src3.at[pl.ds(e0 + w * W, W)], sidx)
        pltpu.sync_copy(dst3.at[pl.ds(e0 + w * W, W)], didx)
        pltpu.sync_copy(msg.at[pl.ds(rev0 + w * W, W)], mbuf)
        pltpu.sync_copy(e_sh.at[sidx], ebuf)

        @plsc.parallel_loop(0, W // 2, unroll=8)
        def pair(i):
            row, col = _pair(i)
            ev = plsc.load_gather(ebuf, [row, col])
            mv = plsc.load_gather(mbuf, [row, col])
            wv = ev / (cc * mv + 1.0)
            nm = wv / _hsum(wv)
            plsc.store_scatter(obuf, [row, col], nm)
            plsc.store_scatter(lbuf, [row, col], _log1p(cc * nm))
        pltpu.sync_copy(obuf, newmsg.at[pl.ds(e0 + w * W, W)])
        pltpu.sync_copy(lbuf, ns_sh.at[didx], add=True)
        return a
    lax.fori_loop(0, NWIN, win, 0)

    plsc.subcore_barrier()
    pltpu.sync_copy(ns_sh.at[pl.ds(n0, NPT)], nsp_out.at[cid].at[pl.ds(n0, NPT)])


def _marg_body(nsp, degp, bovn, psi_out,
               nsA, nsB, dgA, dgB, obuf, htmp, hall, bbuf, h_sh):
    cid = lax.axis_index("c")
    sid = lax.axis_index("s")
    wid = cid * NS + sid
    pltpu.sync_copy(bovn, bbuf)
    it = _it()

    n0 = sid * NPT
    pltpu.sync_copy(nsp.at[0].at[pl.ds(n0, NPT)], nsA)
    pltpu.sync_copy(nsp.at[1].at[pl.ds(n0, NPT)], nsB)
    pltpu.sync_copy(degp.at[0].at[pl.ds(n0, NPT)], dgA)
    pltpu.sync_copy(degp.at[1].at[pl.ds(n0, NPT)], dgB)

    def hstep(i, hacc):
        row, col = _pair(i)
        a = plsc.load_gather(nsA, [row, col]) + plsc.load_gather(nsB, [row, col])
        dv = plsc.load_gather(dgA, [row, col]) + plsc.load_gather(dgB, [row, col])
        p = jnp.exp(a - _hmax(a))
        psi = p / _hsum(p)
        return hacc + dv * psi
    hacc = plsc.parallel_loop(0, NPT // 2, unroll=4,
                              carry=jnp.zeros((16,), F32))(hstep)
    htmp[...] = hacc
    pltpu.sync_copy(htmp, h_sh.at[pl.ds(sid * 16, 16)])
    plsc.subcore_barrier()
    pltpu.sync_copy(h_sh, hall)
    hs = jnp.zeros((16,), F32)
    for k in range(NS):
        hs = hs + hall[pl.ds(k * 16, 16)]
    hs = hs + _perm(hs, it ^ 8)
    hv = bbuf[...] * hs

    g0 = wid * NPO
    pltpu.sync_copy(nsp.at[0].at[pl.ds(g0, NPO)], nsA.at[pl.ds(0, NPO)])
    pltpu.sync_copy(nsp.at[1].at[pl.ds(g0, NPO)], nsB.at[pl.ds(0, NPO)])

    @plsc.parallel_loop(0, NPO // 2, unroll=4)
    def ostep(i):
        row, col = _pair(i)
        v = (plsc.load_gather(nsA, [row, col])
             + plsc.load_gather(nsB, [row, col]) - hv)
        p = jnp.exp(v - _hmax(v))
        psi = p / _hsum(p)
        plsc.store_scatter(obuf, [row, col], psi)
    pltpu.sync_copy(obuf, psi_out.at[pl.ds(g0, NPO)])


def _mesh():
    return plsc.VectorSubcoreMesh(core_axis_name="c", subcore_axis_name="s")


_CPARAMS = pltpu.CompilerParams(needs_layout_passes=False, use_tc_tiling_on_sc=False)


@jax.jit
def _run(msg0, src3, dst3, cvec, bovn):
    k0 = pl.kernel(
        _k0_body,
        out_type=(jax.ShapeDtypeStruct((NC, NP, Q), F32),
                  jax.ShapeDtypeStruct((NC, NP, Q), F32)),
        mesh=_mesh(),
        compiler_params=_CPARAMS,
        scratch_types=[
            pltpu.VMEM((W,), I32),
            pltpu.VMEM((W, Q), F32),
            pltpu.VMEM((W, Q), F32),
            pltpu.VMEM((W, Q), F32),
            pltpu.VMEM((NPT, Q), F32),
            pltpu.VMEM((16,), F32),
            pltpu.VMEM_SHARED((NP, Q), F32),
            pltpu.VMEM_SHARED((NP, Q), F32),
        ],
    )
    gs = pl.kernel(
        _gs_body,
        out_type=(jax.ShapeDtypeStruct((E2, Q), F32),
                  jax.ShapeDtypeStruct((NC, NP, Q), F32)),
        mesh=_mesh(),
        compiler_params=_CPARAMS,
        scratch_types=[
            pltpu.VMEM((W,), I32),
            pltpu.VMEM((W,), I32),
            pltpu.VMEM((W, Q), F32),
            pltpu.VMEM((W, Q), F32),
            pltpu.VMEM((W, Q), F32),
            pltpu.VMEM((W, Q), F32),
            pltpu.VMEM((NPT, Q), F32),
            pltpu.VMEM((NPT, Q), F32),
            pltpu.VMEM((NPT, Q), F32),
            pltpu.VMEM((NPT, Q), F32),
            pltpu.VMEM((NPT, Q), F32),
            pltpu.VMEM((16,), F32),
            pltpu.VMEM((NS * 16,), F32),
            pltpu.VMEM((16,), F32),
            pltpu.VMEM((16,), F32),
            pltpu.VMEM_SHARED((NP, Q), F32),
            pltpu.VMEM_SHARED((NP, Q), F32),
            pltpu.VMEM_SHARED((NS * 16,), F32),
        ],
    )
    marg = pl.kernel(
        _marg_body,
        out_type=jax.ShapeDtypeStruct((NP, Q), F32),
        mesh=_mesh(),
        compiler_params=_CPARAMS,
        scratch_types=[
            pltpu.VMEM((NPT, Q), F32),
            pltpu.VMEM((NPT, Q), F32),
            pltpu.VMEM((NPT, Q), F32),
            pltpu.VMEM((NPT, Q), F32),
            pltpu.VMEM((NPO, Q), F32),
            pltpu.VMEM((16,), F32),
            pltpu.VMEM((NS * 16,), F32),
            pltpu.VMEM((16,), F32),
            pltpu.VMEM_SHARED((NS * 16,), F32),
        ],
    )
    nsp, degp = k0(msg0, dst3, cvec)
    msg = msg0
    for _ in range(5):
        msg, nsp = gs(msg, nsp, degp, src3, dst3, cvec, bovn)
    psi_pad = marg(nsp, degp, bovn)
    return msg, psi_pad[:N_NODES]


def kernel(edge_index, num_nodes, beta, message_map_init):
    src3 = jnp.concatenate([edge_index[0], edge_index[1]])
    dst3 = jnp.concatenate([edge_index[1], edge_index[0]])
    beta = jnp.asarray(beta, F32)
    cvec = jnp.full((16,), jnp.exp(beta) - 1.0, F32)
    bovn = jnp.full((16,), beta / jnp.asarray(num_nodes, F32), F32)
    return _run(message_map_init, src3, dst3, cvec, bovn)
